# baseline (speedup 1.0000x reference)
import functools

import jax
import jax.numpy as jnp
from jax import lax
from jax.experimental import pallas as pl
from jax.experimental.pallas import tpu as pltpu
from jax.experimental.pallas import tpu_sc as plsc

H = 256
K = 204
T_BLK = 4096
B_BLK = 2

_ROWS_STAGE = 256
_STAGE = _ROWS_STAGE * H


def _mask_body(wc_ref, wr_ref, mask_col_ref, mask_row_ref):
    sig_col = jax.nn.sigmoid(wc_ref[...])
    sig_row = jax.nn.sigmoid(wr_ref[...])
    col = jnp.broadcast_to(sig_col, (H, H))
    row = jnp.broadcast_to(sig_row, (H, H))
    cnt_col = jnp.sum((row > col).astype(jnp.float32), axis=1, keepdims=True)
    mask_col_ref[...] = (cnt_col < float(K)).astype(jnp.float32)
    cnt_row = jnp.sum((col > row).astype(jnp.float32), axis=0, keepdims=True)
    mask_row_ref[...] = (cnt_row < float(K)).astype(jnp.float32)


def _mx_body(x_ref, mask_col_ref, mx_ref):
    mask_col = mask_col_ref[...]
    mx_ref[...] = x_ref[...] * mask_col[None, None]


def _bm_writer(nc, mask_hbm, out_hbm, mask_v, stage_v, sem):
    cid = lax.axis_index("c")
    sid = lax.axis_index("s")
    wid = sid * nc + cid
    pltpu.sync_copy(mask_hbm, mask_v)
    vregs = [mask_v[pl.ds(k * 16, 16)] for k in range(H // 16)]

    def fill_row(r, off):
        for k in range(H // 16):
            stage_v[pl.ds(off + k * 16, 16)] = vregs[k]
        return off + H

    lax.fori_loop(0, _ROWS_STAGE, fill_row, 0)

    n_chunks = out_hbm.shape[1]
    handles = [pltpu.async_copy(stage_v, out_hbm.at[wid, ch], sem)
               for ch in range(n_chunks)]
    for h in handles:
        h.wait()


def kernel(x, W):
    b, c, nfreq, ntime = x.shape

    mask_col, mask_row = pl.pallas_call(
        _mask_body,
        out_shape=(
            jax.ShapeDtypeStruct((H, 1), jnp.float32),
            jax.ShapeDtypeStruct((1, H), jnp.float32),
        ),
    )(W.reshape(H, 1), W.reshape(1, H))

    info = plsc.get_sparse_core_info()
    mesh = plsc.VectorSubcoreMesh(core_axis_name="c", subcore_axis_name="s")
    n_chunks = ntime * H // _STAGE
    bm_writer = functools.partial(
        pl.kernel,
        out_type=jax.ShapeDtypeStruct((b, n_chunks, _STAGE), jnp.float32),
        mesh=mesh,
        scratch_types=[
            pltpu.VMEM((H,), jnp.float32),
            pltpu.VMEM((_STAGE,), jnp.float32),
            pltpu.SemaphoreType.DMA,
        ],
    )(functools.partial(_bm_writer, info.num_cores))
    binary_mask = bm_writer(mask_row.reshape(H)).reshape(b, c, ntime, nfreq)

    masked_x = pl.pallas_call(
        _mx_body,
        grid=(b // B_BLK, ntime // T_BLK),
        in_specs=[
            pl.BlockSpec((B_BLK, 1, nfreq, T_BLK), lambda i, j: (i, 0, 0, j)),
            pl.BlockSpec((H, 1), lambda i, j: (0, 0)),
        ],
        out_specs=pl.BlockSpec((B_BLK, 1, nfreq, T_BLK),
                               lambda i, j: (i, 0, 0, j)),
        out_shape=jax.ShapeDtypeStruct((b, c, nfreq, ntime), x.dtype),
        compiler_params=pltpu.CompilerParams(
            dimension_semantics=("parallel", "parallel"),
        ),
    )(x, mask_col)

    return masked_x, binary_mask


# speedup vs baseline: 2.1139x; 2.1139x over previous
import functools

import jax
import jax.numpy as jnp
from jax.experimental import pallas as pl
from jax.experimental.pallas import tpu as pltpu

H = 256
K = 204
T_BLK = 2048
B_BLK = 4


def _fused_body(x_ref, wc_ref, wr_ref, mx_ref, bm_ref):
    sig_col = jax.nn.sigmoid(wc_ref[...])
    sig_row = jax.nn.sigmoid(wr_ref[...])
    col = jnp.broadcast_to(sig_col, (H, H))
    row = jnp.broadcast_to(sig_row, (H, H))
    cnt_col = jnp.sum((row > col).astype(jnp.float32), axis=1, keepdims=True)
    mask_col = (cnt_col < float(K)).astype(jnp.float32)
    cnt_row = jnp.sum((col > row).astype(jnp.float32), axis=0, keepdims=True)
    mask_row = (cnt_row < float(K)).astype(jnp.float32)

    mx_ref[...] = x_ref[...] * mask_col[None, None]
    bm_ref[...] = jnp.broadcast_to(mask_row[None, None], (B_BLK, 1, T_BLK, H))


def kernel(x, W):
    b, c, nfreq, ntime = x.shape

    grid = (b // B_BLK, ntime // T_BLK)
    masked_x, binary_mask = pl.pallas_call(
        _fused_body,
        grid=grid,
        in_specs=[
            pl.BlockSpec((B_BLK, 1, nfreq, T_BLK), lambda i, j: (i, 0, 0, j)),
            pl.BlockSpec((H, 1), lambda i, j: (0, 0)),
            pl.BlockSpec((1, H), lambda i, j: (0, 0)),
        ],
        out_specs=(
            pl.BlockSpec((B_BLK, 1, nfreq, T_BLK), lambda i, j: (i, 0, 0, j)),
            pl.BlockSpec((B_BLK, 1, T_BLK, H), lambda i, j: (i, 0, j, 0)),
        ),
        out_shape=(
            jax.ShapeDtypeStruct((b, c, nfreq, ntime), x.dtype),
            jax.ShapeDtypeStruct((b, c, ntime, nfreq), x.dtype),
        ),
        compiler_params=pltpu.CompilerParams(
            dimension_semantics=("parallel", "parallel"),
        ),
    )(x, W.reshape(H, 1), W.reshape(1, H))

    return masked_x, binary_mask
